# Initial kernel scaffold; baseline (speedup 1.0000x reference)
import functools

import jax
import jax.numpy as jnp
from jax import lax
from jax.experimental import pallas as pl
from jax.experimental.pallas import tpu as pltpu
from jax.experimental.pallas import tpu_sc as plsc

N = 10000
E = 160000
F = 256
H = 256
C = 40

NC, NS = 2, 16
RPT = N // NS

K1, CH1 = 125, 80
K2, CH2 = 125, 40

RB = 1000
GR = N // RB

_MESH = plsc.VectorSubcoreMesh(
    core_axis_name="c", subcore_axis_name="s", num_cores=NC, num_subcores=NS)


@functools.partial(
    pl.kernel,
    out_type=jax.ShapeDtypeStruct((NC, N), jnp.float32),
    mesh=_MESH,
    scratch_types=[
        pltpu.VMEM((CH2, K2), jnp.int32),
        pltpu.VMEM((K2,), jnp.float32),
        pltpu.VMEM((N,), jnp.float32),
        pltpu.VMEM_SHARED((N,), jnp.float32),
    ],
)
def _deg_kernel(dst2_hbm, ones_hbm, zeros_hbm, deg_out, idx_v, ones_v, stage_v,
                deg_sh):
    c = lax.axis_index("c")
    s = lax.axis_index("s")

    @pl.when(s == 0)
    def _():
        pltpu.sync_copy(zeros_hbm, stage_v)
        pltpu.sync_copy(stage_v, deg_sh)

    pltpu.sync_copy(dst2_hbm.at[c, s], idx_v)
    pltpu.sync_copy(ones_hbm, ones_v)
    plsc.subcore_barrier()

    @pl.loop(0, CH2)
    def _(j):
        pltpu.sync_copy(ones_v, deg_sh.at[idx_v.at[j]], add=True)

    plsc.subcore_barrier()

    @pl.when(s == 0)
    def _():
        pltpu.sync_copy(deg_sh, stage_v)
        pltpu.sync_copy(stage_v, deg_out.at[c])


@functools.partial(
    pl.kernel,
    out_type=jax.ShapeDtypeStruct((NC * N, H // 2), jnp.float32),
    mesh=_MESH,
    scratch_types=[
        pltpu.VMEM((CH1, K1), jnp.int32),
        pltpu.VMEM((CH1, K1), jnp.int32),
        pltpu.VMEM((K1, H // 2), jnp.float32),
        pltpu.VMEM((RPT, H // 2), jnp.float32),
        pltpu.VMEM_SHARED((N, H // 2), jnp.float32),
        pltpu.SemaphoreType.DMA,
    ],
)
def _layer1_kernel(y_hbm, src1_hbm, dst1_hbm, acc_out, sidx_v, didx_v, rows_v,
                   stage_v, acc_sh, sem):
    c = lax.axis_index("c")
    s = lax.axis_index("s")
    row0 = c * N + s * RPT

    pltpu.sync_copy(y_hbm.at[pl.ds(row0, RPT)], stage_v)
    pltpu.sync_copy(stage_v, acc_sh.at[pl.ds(s * RPT, RPT)])
    pltpu.sync_copy(src1_hbm.at[c, s], sidx_v)
    pltpu.sync_copy(dst1_hbm.at[s], didx_v)
    plsc.subcore_barrier()

    @pl.loop(0, CH1)
    def _(j):
        pltpu.async_copy(y_hbm.at[sidx_v.at[j]], rows_v, sem).wait()
        pltpu.sync_copy(rows_v, acc_sh.at[didx_v.at[j]], add=True)

    plsc.subcore_barrier()
    pltpu.sync_copy(acc_sh.at[pl.ds(s * RPT, RPT)], stage_v)
    pltpu.sync_copy(stage_v, acc_out.at[pl.ds(row0, RPT)])


@functools.partial(
    pl.kernel,
    out_type=jax.ShapeDtypeStruct((NC * N, C), jnp.float32),
    mesh=_MESH,
    scratch_types=[
        pltpu.VMEM((CH2, K2), jnp.int32),
        pltpu.VMEM((CH2, K2), jnp.int32),
        pltpu.VMEM((K2, C), jnp.float32),
        pltpu.VMEM((RPT, C), jnp.float32),
        pltpu.VMEM_SHARED((N, C), jnp.float32),
        pltpu.SemaphoreType.DMA,
    ],
)
def _layer2_kernel(y2_hbm, src2_hbm, dst2_hbm, acc_out, sidx_v, didx_v, rows_v,
                   stage_v, acc_sh, sem):
    c = lax.axis_index("c")
    s = lax.axis_index("s")
    row0 = c * N + s * RPT

    pltpu.sync_copy(y2_hbm.at[pl.ds(row0, RPT)], stage_v)
    pltpu.sync_copy(stage_v, acc_sh.at[pl.ds(s * RPT, RPT)])
    pltpu.sync_copy(src2_hbm.at[c, s], sidx_v)
    pltpu.sync_copy(dst2_hbm.at[c, s], didx_v)
    plsc.subcore_barrier()

    @pl.loop(0, CH2)
    def _(j):
        pltpu.async_copy(y2_hbm.at[sidx_v.at[j]], rows_v, sem).wait()
        pltpu.sync_copy(rows_v, acc_sh.at[didx_v.at[j]], add=True)

    plsc.subcore_barrier()
    pltpu.sync_copy(acc_sh.at[pl.ds(s * RPT, RPT)], stage_v)
    pltpu.sync_copy(stage_v, acc_out.at[pl.ds(row0, RPT)])


def _mm1_body(x_ref, w_ref, degp_ref, y_ref):
    dinv = lax.rsqrt(degp_ref[0] + degp_ref[1] + 1.0)
    xw = jnp.dot(x_ref[...], w_ref[...], preferred_element_type=jnp.float32)
    y_ref[...] = xw * dinv


def _mm2_body(acc_ref, degp_ref, b1_ref, w2_ref, y2_ref):
    dinv = lax.rsqrt(degp_ref[0] + degp_ref[1] + 1.0)
    h_l = jnp.maximum(acc_ref[0] * dinv + b1_ref[:, : H // 2], 0.0)
    h_r = jnp.maximum(acc_ref[1] * dinv + b1_ref[:, H // 2:], 0.0)
    hw = (jnp.dot(h_l, w2_ref[: H // 2, :], preferred_element_type=jnp.float32)
          + jnp.dot(h_r, w2_ref[H // 2:, :], preferred_element_type=jnp.float32))
    y2 = hw * dinv
    y2_ref[0] = y2
    y2_ref[1] = jnp.zeros_like(y2)


def _final_body(acc_ref, degp_ref, b2_ref, out_ref):
    dinv = lax.rsqrt(degp_ref[0] + degp_ref[1] + 1.0)
    z = (acc_ref[0] + acc_ref[1]) * dinv + b2_ref[:, :]
    m = jnp.max(z, axis=1, keepdims=True)
    zm = z - m
    lse = jnp.log(jnp.sum(jnp.exp(zm), axis=1, keepdims=True))
    out_ref[...] = zm - lse


def kernel(x, edge_index, W1, b1, W2, b2):
    src = edge_index[0].astype(jnp.int32)
    dst = edge_index[1].astype(jnp.int32)

    src1 = src.reshape(NS, CH1, K1)
    src1 = jnp.stack([src1, src1 + N])
    dst1 = dst.reshape(NS, CH1, K1)
    src2 = src.reshape(NC, NS, CH2, K2)
    dst2 = dst.reshape(NC, NS, CH2, K2)

    ones_k = jnp.ones((K2,), jnp.float32)
    zeros_n = jnp.zeros((N,), jnp.float32)

    deg_p = _deg_kernel(dst2, ones_k, zeros_n)
    degp_r = deg_p.reshape(NC, N, 1)

    y1 = pl.pallas_call(
        _mm1_body,
        grid=(NC, GR),
        in_specs=[
            pl.BlockSpec((RB, F), lambda j, i: (i, 0)),
            pl.BlockSpec((F, H // 2), lambda j, i: (0, j)),
            pl.BlockSpec((NC, RB, 1), lambda j, i: (0, i, 0)),
        ],
        out_specs=pl.BlockSpec((RB, H // 2), lambda j, i: (j * GR + i, 0)),
        out_shape=jax.ShapeDtypeStruct((NC * N, H // 2), jnp.float32),
    )(x, W1, degp_r)

    acc1 = _layer1_kernel(y1, src1, dst1)
    acc1_r = acc1.reshape(NC, N, H // 2)

    y2i = pl.pallas_call(
        _mm2_body,
        grid=(GR,),
        in_specs=[
            pl.BlockSpec((NC, RB, H // 2), lambda i: (0, i, 0)),
            pl.BlockSpec((NC, RB, 1), lambda i: (0, i, 0)),
            pl.BlockSpec((1, H), lambda i: (0, 0)),
            pl.BlockSpec((H, C), lambda i: (0, 0)),
        ],
        out_specs=pl.BlockSpec((NC, RB, C), lambda i: (0, i, 0)),
        out_shape=jax.ShapeDtypeStruct((NC, N, C), jnp.float32),
    )(acc1_r, degp_r, b1.reshape(1, H), W2)

    acc2 = _layer2_kernel(y2i.reshape(NC * N, C), src2, dst2)
    acc2_r = acc2.reshape(NC, N, C)

    out = pl.pallas_call(
        _final_body,
        grid=(GR,),
        in_specs=[
            pl.BlockSpec((NC, RB, C), lambda i: (0, i, 0)),
            pl.BlockSpec((NC, RB, 1), lambda i: (0, i, 0)),
            pl.BlockSpec((1, C), lambda i: (0, 0)),
        ],
        out_specs=pl.BlockSpec((RB, C), lambda i: (i, 0)),
        out_shape=jax.ShapeDtypeStruct((N, C), jnp.float32),
    )(acc2_r, degp_r, b2.reshape(1, C))
    return out


# speedup vs baseline: 15.5605x; 15.5605x over previous
import functools

import jax
import jax.numpy as jnp
from jax import lax
from jax.experimental import pallas as pl
from jax.experimental.pallas import tpu as pltpu
from jax.experimental.pallas import tpu_sc as plsc

N = 10000
E = 160000
F = 256
H = 256
C = 40

NC, NS = 2, 16
RPT = N // NS

K1, CH1 = 125, 80
K2, CH2 = 125, 40

RB = 1000
GR = N // RB

_MESH = plsc.VectorSubcoreMesh(
    core_axis_name="c", subcore_axis_name="s", num_cores=NC, num_subcores=NS)


@functools.partial(
    pl.kernel,
    out_type=jax.ShapeDtypeStruct((NC, 1, N), jnp.float32),
    mesh=_MESH,
    compiler_params=pltpu.CompilerParams(use_tc_tiling_on_sc=False),
    scratch_types=[
        pltpu.VMEM((CH2, K2), jnp.int32),
        pltpu.VMEM((K2,), jnp.float32),
        pltpu.VMEM((N,), jnp.float32),
        pltpu.VMEM_SHARED((N,), jnp.float32),
    ],
)
def _deg_kernel(dst2_hbm, ones_hbm, zeros_hbm, deg_out, idx_v, ones_v, stage_v,
                deg_sh):
    c = lax.axis_index("c")
    s = lax.axis_index("s")

    @pl.when(s == 0)
    def _():
        pltpu.sync_copy(zeros_hbm, stage_v)
        pltpu.sync_copy(stage_v, deg_sh)

    pltpu.sync_copy(dst2_hbm.at[c, s], idx_v)
    pltpu.sync_copy(ones_hbm, ones_v)
    plsc.subcore_barrier()

    @pl.loop(0, CH2)
    def _(j):
        pltpu.sync_copy(ones_v, deg_sh.at[idx_v.at[j]], add=True)

    plsc.subcore_barrier()

    @pl.when(s == 0)
    def _():
        pltpu.sync_copy(deg_sh, stage_v)
        pltpu.sync_copy(stage_v, deg_out.at[c, 0])


@functools.partial(
    pl.kernel,
    out_type=jax.ShapeDtypeStruct((NC * N, H // 2), jnp.float32),
    mesh=_MESH,
    compiler_params=pltpu.CompilerParams(use_tc_tiling_on_sc=False),
    scratch_types=[
        pltpu.VMEM((CH1, K1), jnp.int32),
        pltpu.VMEM((CH1, K1), jnp.int32),
        pltpu.VMEM((K1, H // 2), jnp.float32),
        pltpu.VMEM_SHARED((N, H // 2), jnp.float32),
        pltpu.SemaphoreType.DMA,
    ],
)
def _layer1_kernel(y_hbm, src1_hbm, dst1_hbm, acc_out, sidx_v, didx_v, rows_v,
                   acc_sh, sem):
    c = lax.axis_index("c")
    s = lax.axis_index("s")

    @pl.when(s == 0)
    def _():
        pltpu.sync_copy(y_hbm.at[pl.ds(c * N, N)], acc_sh)

    pltpu.sync_copy(src1_hbm.at[c, s], sidx_v)
    pltpu.sync_copy(dst1_hbm.at[s], didx_v)
    plsc.subcore_barrier()

    @pl.loop(0, CH1)
    def _(j):
        pltpu.async_copy(y_hbm.at[sidx_v.at[j]], rows_v, sem).wait()
        pltpu.sync_copy(rows_v, acc_sh.at[didx_v.at[j]], add=True)

    plsc.subcore_barrier()

    @pl.when(s == 0)
    def _():
        pltpu.sync_copy(acc_sh, acc_out.at[pl.ds(c * N, N)])


@functools.partial(
    pl.kernel,
    out_type=jax.ShapeDtypeStruct((NC * N, C), jnp.float32),
    mesh=_MESH,
    compiler_params=pltpu.CompilerParams(use_tc_tiling_on_sc=False),
    scratch_types=[
        pltpu.VMEM((CH2, K2), jnp.int32),
        pltpu.VMEM((CH2, K2), jnp.int32),
        pltpu.VMEM((K2, C), jnp.float32),
        pltpu.VMEM_SHARED((N, C), jnp.float32),
        pltpu.SemaphoreType.DMA,
    ],
)
def _layer2_kernel(y2_hbm, src2_hbm, dst2_hbm, acc_out, sidx_v, didx_v, rows_v,
                   acc_sh, sem):
    c = lax.axis_index("c")
    s = lax.axis_index("s")

    @pl.when(s == 0)
    def _():
        pltpu.sync_copy(y2_hbm.at[pl.ds(c * N, N)], acc_sh)

    pltpu.sync_copy(src2_hbm.at[c, s], sidx_v)
    pltpu.sync_copy(dst2_hbm.at[c, s], didx_v)
    plsc.subcore_barrier()

    @pl.loop(0, CH2)
    def _(j):
        pltpu.async_copy(y2_hbm.at[sidx_v.at[j]], rows_v, sem).wait()
        pltpu.sync_copy(rows_v, acc_sh.at[didx_v.at[j]], add=True)

    plsc.subcore_barrier()

    @pl.when(s == 0)
    def _():
        pltpu.sync_copy(acc_sh, acc_out.at[pl.ds(c * N, N)])


def _mm1_body(x_ref, w_ref, degp_ref, y_ref):
    dinv = lax.rsqrt(degp_ref[0] + degp_ref[1] + 1.0)
    xw = jnp.dot(x_ref[...], w_ref[...], preferred_element_type=jnp.float32)
    y_ref[...] = xw * dinv


def _mm2_body(acc_ref, degp_ref, b1_ref, w2_ref, y2_ref):
    dinv = lax.rsqrt(degp_ref[0] + degp_ref[1] + 1.0)
    h_l = jnp.maximum(acc_ref[0] * dinv + b1_ref[:, : H // 2], 0.0)
    h_r = jnp.maximum(acc_ref[1] * dinv + b1_ref[:, H // 2:], 0.0)
    hw = (jnp.dot(h_l, w2_ref[: H // 2, :], preferred_element_type=jnp.float32)
          + jnp.dot(h_r, w2_ref[H // 2:, :], preferred_element_type=jnp.float32))
    y2 = hw * dinv
    y2_ref[0] = y2
    y2_ref[1] = jnp.zeros_like(y2)


def _final_body(acc_ref, degp_ref, b2_ref, out_ref):
    dinv = lax.rsqrt(degp_ref[0] + degp_ref[1] + 1.0)
    z = (acc_ref[0] + acc_ref[1]) * dinv + b2_ref[:, :]
    m = jnp.max(z, axis=1, keepdims=True)
    zm = z - m
    lse = jnp.log(jnp.sum(jnp.exp(zm), axis=1, keepdims=True))
    out_ref[...] = zm - lse


def kernel(x, edge_index, W1, b1, W2, b2):
    src = edge_index[0].astype(jnp.int32)
    dst = edge_index[1].astype(jnp.int32)

    src1 = src.reshape(NS, CH1, K1)
    src1 = jnp.stack([src1, src1 + N])
    dst1 = dst.reshape(NS, CH1, K1)
    src2 = src.reshape(NC, NS, CH2, K2)
    dst2 = dst.reshape(NC, NS, CH2, K2)

    ones_k = jnp.ones((K2,), jnp.float32)
    zeros_n = jnp.zeros((N,), jnp.float32)

    deg_p = _deg_kernel(dst2, ones_k, zeros_n)
    degp_r = deg_p.reshape(NC, N, 1)

    y1 = pl.pallas_call(
        _mm1_body,
        grid=(NC, GR),
        in_specs=[
            pl.BlockSpec((RB, F), lambda j, i: (i, 0)),
            pl.BlockSpec((F, H // 2), lambda j, i: (0, j)),
            pl.BlockSpec((NC, RB, 1), lambda j, i: (0, i, 0)),
        ],
        out_specs=pl.BlockSpec((RB, H // 2), lambda j, i: (j * GR + i, 0)),
        out_shape=jax.ShapeDtypeStruct((NC * N, H // 2), jnp.float32),
    )(x, W1, degp_r)

    acc1 = _layer1_kernel(y1, src1, dst1)
    acc1_r = acc1.reshape(NC, N, H // 2)

    y2i = pl.pallas_call(
        _mm2_body,
        grid=(GR,),
        in_specs=[
            pl.BlockSpec((NC, RB, H // 2), lambda i: (0, i, 0)),
            pl.BlockSpec((NC, RB, 1), lambda i: (0, i, 0)),
            pl.BlockSpec((1, H), lambda i: (0, 0)),
            pl.BlockSpec((H, C), lambda i: (0, 0)),
        ],
        out_specs=pl.BlockSpec((NC, RB, C), lambda i: (0, i, 0)),
        out_shape=jax.ShapeDtypeStruct((NC, N, C), jnp.float32),
    )(acc1_r, degp_r, b1.reshape(1, H), W2)

    acc2 = _layer2_kernel(y2i.reshape(NC * N, C), src2, dst2)
    acc2_r = acc2.reshape(NC, N, C)

    out = pl.pallas_call(
        _final_body,
        grid=(GR,),
        in_specs=[
            pl.BlockSpec((NC, RB, C), lambda i: (0, i, 0)),
            pl.BlockSpec((NC, RB, 1), lambda i: (0, i, 0)),
            pl.BlockSpec((1, C), lambda i: (0, 0)),
        ],
        out_specs=pl.BlockSpec((RB, C), lambda i: (i, 0)),
        out_shape=jax.ShapeDtypeStruct((N, C), jnp.float32),
    )(acc2_r, degp_r, b2.reshape(1, C))
    return out


# speedup vs baseline: 21.1918x; 1.3619x over previous
import functools

import jax
import jax.numpy as jnp
from jax import lax
from jax.experimental import pallas as pl
from jax.experimental.pallas import tpu as pltpu
from jax.experimental.pallas import tpu_sc as plsc

N = 10000
E = 160000
F = 256
H = 256
C = 40

NC, NS = 2, 16
RPT = N // NS

K1, CH1 = 100, 100
K2, CH2 = 125, 40

NB1 = 2
NB2 = 4

RB = 1000
GR = N // RB

_MESH = plsc.VectorSubcoreMesh(
    core_axis_name="c", subcore_axis_name="s", num_cores=NC, num_subcores=NS)


@functools.partial(
    pl.kernel,
    out_type=jax.ShapeDtypeStruct((NC, 1, N), jnp.float32),
    mesh=_MESH,
    compiler_params=pltpu.CompilerParams(use_tc_tiling_on_sc=False),
    scratch_types=[
        pltpu.VMEM((CH2, K2), jnp.int32),
        pltpu.VMEM((K2,), jnp.float32),
        pltpu.VMEM((N,), jnp.float32),
        pltpu.VMEM_SHARED((N,), jnp.float32),
    ],
)
def _deg_kernel(dst2_hbm, ones_hbm, zeros_hbm, deg_out, idx_v, ones_v, stage_v,
                deg_sh):
    c = lax.axis_index("c")
    s = lax.axis_index("s")

    @pl.when(s == 0)
    def _():
        pltpu.sync_copy(zeros_hbm, stage_v)
        pltpu.sync_copy(stage_v, deg_sh)

    pltpu.sync_copy(dst2_hbm.at[c, s], idx_v)
    pltpu.sync_copy(ones_hbm, ones_v)
    plsc.subcore_barrier()

    @pl.loop(0, CH2)
    def _(j):
        pltpu.sync_copy(ones_v, deg_sh.at[idx_v.at[j]], add=True)

    plsc.subcore_barrier()

    @pl.when(s == 0)
    def _():
        pltpu.sync_copy(deg_sh, stage_v)
        pltpu.sync_copy(stage_v, deg_out.at[c, 0])


@functools.partial(
    pl.kernel,
    out_type=jax.ShapeDtypeStruct((NC * N, H // 2), jnp.float32),
    mesh=_MESH,
    compiler_params=pltpu.CompilerParams(use_tc_tiling_on_sc=False),
    scratch_types=[
        pltpu.VMEM((CH1, K1), jnp.int32),
        pltpu.VMEM((CH1, K1), jnp.int32),
        pltpu.VMEM((K1, H // 2), jnp.float32),
        pltpu.VMEM((K1, H // 2), jnp.float32),
        pltpu.SemaphoreType.DMA,
        pltpu.SemaphoreType.DMA,
        pltpu.SemaphoreType.DMA,
        pltpu.SemaphoreType.DMA,
        pltpu.VMEM_SHARED((N, H // 2), jnp.float32),
    ],
)
def _layer1_kernel(y_hbm, src1_hbm, dst1_hbm, acc_out, sidx_v, didx_v,
                   r0, r1, g0, g1, s0, s1, acc_sh):
    rows_v = (r0, r1)
    gsem = (g0, g1)
    ssem = (s0, s1)
    c = lax.axis_index("c")
    s = lax.axis_index("s")

    @pl.when(s == 0)
    def _():
        pltpu.sync_copy(y_hbm.at[pl.ds(c * N, N)], acc_sh)

    pltpu.sync_copy(src1_hbm.at[c, s], sidx_v)
    pltpu.sync_copy(dst1_hbm.at[s], didx_v)
    plsc.subcore_barrier()

    for b in range(NB1):
        pltpu.async_copy(y_hbm.at[sidx_v.at[b]], rows_v[b], gsem[b])

    @pl.loop(0, CH1, step=NB1)
    def _(j):
        for b in range(NB1):
            i = j + b
            pltpu.make_async_copy(y_hbm.at[sidx_v.at[i]], rows_v[b],
                                  gsem[b]).wait()
            pltpu.async_copy(rows_v[b], acc_sh.at[didx_v.at[i]], ssem[b],
                             add=True)

            @pl.when(i + NB1 < CH1)
            def _():
                pltpu.make_async_copy(rows_v[b], acc_sh.at[didx_v.at[i]],
                                      ssem[b]).wait()
                pltpu.async_copy(y_hbm.at[sidx_v.at[i + NB1]], rows_v[b],
                                 gsem[b])

    for b in range(NB1):
        pltpu.make_async_copy(rows_v[b], acc_sh.at[didx_v.at[b]],
                              ssem[b]).wait()

    plsc.subcore_barrier()

    @pl.when(s == 0)
    def _():
        pltpu.sync_copy(acc_sh, acc_out.at[pl.ds(c * N, N)])


@functools.partial(
    pl.kernel,
    out_type=jax.ShapeDtypeStruct((NC * N, C), jnp.float32),
    mesh=_MESH,
    compiler_params=pltpu.CompilerParams(use_tc_tiling_on_sc=False),
    scratch_types=[
        pltpu.VMEM((CH2, K2), jnp.int32),
        pltpu.VMEM((CH2, K2), jnp.int32),
        pltpu.VMEM((K2, C), jnp.float32),
        pltpu.VMEM((K2, C), jnp.float32),
        pltpu.VMEM((K2, C), jnp.float32),
        pltpu.VMEM((K2, C), jnp.float32),
        pltpu.SemaphoreType.DMA,
        pltpu.SemaphoreType.DMA,
        pltpu.SemaphoreType.DMA,
        pltpu.SemaphoreType.DMA,
        pltpu.SemaphoreType.DMA,
        pltpu.SemaphoreType.DMA,
        pltpu.SemaphoreType.DMA,
        pltpu.SemaphoreType.DMA,
        pltpu.VMEM_SHARED((N, C), jnp.float32),
    ],
)
def _layer2_kernel(y2_hbm, src2_hbm, dst2_hbm, acc_out, sidx_v, didx_v,
                   r0, r1, r2, r3, g0, g1, g2, g3, s0, s1, s2, s3, acc_sh):
    rows_v = (r0, r1, r2, r3)
    gsem = (g0, g1, g2, g3)
    ssem = (s0, s1, s2, s3)
    c = lax.axis_index("c")
    s = lax.axis_index("s")

    @pl.when(s == 0)
    def _():
        pltpu.sync_copy(y2_hbm.at[pl.ds(c * N, N)], acc_sh)

    pltpu.sync_copy(src2_hbm.at[c, s], sidx_v)
    pltpu.sync_copy(dst2_hbm.at[c, s], didx_v)
    plsc.subcore_barrier()

    for b in range(NB2):
        pltpu.async_copy(y2_hbm.at[sidx_v.at[b]], rows_v[b], gsem[b])

    @pl.loop(0, CH2, step=NB2)
    def _(j):
        for b in range(NB2):
            i = j + b
            pltpu.make_async_copy(y2_hbm.at[sidx_v.at[i]], rows_v[b],
                                  gsem[b]).wait()
            pltpu.async_copy(rows_v[b], acc_sh.at[didx_v.at[i]], ssem[b],
                             add=True)

            @pl.when(i + NB2 < CH2)
            def _():
                pltpu.make_async_copy(rows_v[b], acc_sh.at[didx_v.at[i]],
                                      ssem[b]).wait()
                pltpu.async_copy(y2_hbm.at[sidx_v.at[i + NB2]], rows_v[b],
                                 gsem[b])

    for b in range(NB2):
        pltpu.make_async_copy(rows_v[b], acc_sh.at[didx_v.at[b]],
                              ssem[b]).wait()

    plsc.subcore_barrier()

    @pl.when(s == 0)
    def _():
        pltpu.sync_copy(acc_sh, acc_out.at[pl.ds(c * N, N)])


def _mm1_body(x_ref, w_ref, degp_ref, y_ref):
    dinv = lax.rsqrt(degp_ref[0] + degp_ref[1] + 1.0)
    xw = jnp.dot(x_ref[...], w_ref[...], preferred_element_type=jnp.float32)
    y_ref[...] = xw * dinv


def _mm2_body(acc_ref, degp_ref, b1_ref, w2_ref, y2_ref):
    dinv = lax.rsqrt(degp_ref[0] + degp_ref[1] + 1.0)
    h_l = jnp.maximum(acc_ref[0] * dinv + b1_ref[:, : H // 2], 0.0)
    h_r = jnp.maximum(acc_ref[1] * dinv + b1_ref[:, H // 2:], 0.0)
    hw = (jnp.dot(h_l, w2_ref[: H // 2, :], preferred_element_type=jnp.float32)
          + jnp.dot(h_r, w2_ref[H // 2:, :], preferred_element_type=jnp.float32))
    y2 = hw * dinv
    y2_ref[0] = y2
    y2_ref[1] = jnp.zeros_like(y2)


def _final_body(acc_ref, degp_ref, b2_ref, out_ref):
    dinv = lax.rsqrt(degp_ref[0] + degp_ref[1] + 1.0)
    z = (acc_ref[0] + acc_ref[1]) * dinv + b2_ref[:, :]
    m = jnp.max(z, axis=1, keepdims=True)
    zm = z - m
    lse = jnp.log(jnp.sum(jnp.exp(zm), axis=1, keepdims=True))
    out_ref[...] = zm - lse


def kernel(x, edge_index, W1, b1, W2, b2):
    src = edge_index[0].astype(jnp.int32)
    dst = edge_index[1].astype(jnp.int32)

    src1 = src.reshape(NS, CH1, K1)
    src1 = jnp.stack([src1, src1 + N])
    dst1 = dst.reshape(NS, CH1, K1)
    src2 = src.reshape(NC, NS, CH2, K2)
    dst2 = dst.reshape(NC, NS, CH2, K2)

    ones_k = jnp.ones((K2,), jnp.float32)
    zeros_n = jnp.zeros((N,), jnp.float32)

    deg_p = _deg_kernel(dst2, ones_k, zeros_n)
    degp_r = deg_p.reshape(NC, N, 1)

    y1 = pl.pallas_call(
        _mm1_body,
        grid=(NC, GR),
        in_specs=[
            pl.BlockSpec((RB, F), lambda j, i: (i, 0)),
            pl.BlockSpec((F, H // 2), lambda j, i: (0, j)),
            pl.BlockSpec((NC, RB, 1), lambda j, i: (0, i, 0)),
        ],
        out_specs=pl.BlockSpec((RB, H // 2), lambda j, i: (j * GR + i, 0)),
        out_shape=jax.ShapeDtypeStruct((NC * N, H // 2), jnp.float32),
    )(x, W1, degp_r)

    acc1 = _layer1_kernel(y1, src1, dst1)
    acc1_r = acc1.reshape(NC, N, H // 2)

    y2i = pl.pallas_call(
        _mm2_body,
        grid=(GR,),
        in_specs=[
            pl.BlockSpec((NC, RB, H // 2), lambda i: (0, i, 0)),
            pl.BlockSpec((NC, RB, 1), lambda i: (0, i, 0)),
            pl.BlockSpec((1, H), lambda i: (0, 0)),
            pl.BlockSpec((H, C), lambda i: (0, 0)),
        ],
        out_specs=pl.BlockSpec((NC, RB, C), lambda i: (0, i, 0)),
        out_shape=jax.ShapeDtypeStruct((NC, N, C), jnp.float32),
    )(acc1_r, degp_r, b1.reshape(1, H), W2)

    acc2 = _layer2_kernel(y2i.reshape(NC * N, C), src2, dst2)
    acc2_r = acc2.reshape(NC, N, C)

    out = pl.pallas_call(
        _final_body,
        grid=(GR,),
        in_specs=[
            pl.BlockSpec((NC, RB, C), lambda i: (0, i, 0)),
            pl.BlockSpec((NC, RB, 1), lambda i: (0, i, 0)),
            pl.BlockSpec((1, C), lambda i: (0, 0)),
        ],
        out_specs=pl.BlockSpec((RB, C), lambda i: (i, 0)),
        out_shape=jax.ShapeDtypeStruct((N, C), jnp.float32),
    )(acc2_r, degp_r, b2.reshape(1, C))
    return out


# speedup vs baseline: 22.0822x; 1.0420x over previous
import functools

import jax
import jax.numpy as jnp
from jax import lax
from jax.experimental import pallas as pl
from jax.experimental.pallas import tpu as pltpu
from jax.experimental.pallas import tpu_sc as plsc

N = 10000
E = 160000
F = 256
H = 256
C = 40

NC, NS = 2, 16
RPT = N // NS

K1, CH1 = 100, 100
K2, CH2 = 125, 40

NB1 = 2
NB2 = 4

RB = 1000
GR = N // RB

_MESH = plsc.VectorSubcoreMesh(
    core_axis_name="c", subcore_axis_name="s", num_cores=NC, num_subcores=NS)


@functools.partial(
    pl.kernel,
    out_type=jax.ShapeDtypeStruct((NC, 1, N), jnp.float32),
    mesh=_MESH,
    compiler_params=pltpu.CompilerParams(use_tc_tiling_on_sc=False),
    scratch_types=[
        pltpu.VMEM((CH2, K2), jnp.int32),
        pltpu.VMEM((K2,), jnp.float32),
        pltpu.VMEM((N,), jnp.float32),
        pltpu.VMEM_SHARED((N,), jnp.float32),
    ],
)
def _deg_kernel(dst2_hbm, ones_hbm, zeros_hbm, deg_out, idx_v, ones_v, stage_v,
                deg_sh):
    c = lax.axis_index("c")
    s = lax.axis_index("s")

    @pl.when(s == 0)
    def _():
        pltpu.sync_copy(zeros_hbm, stage_v)
        pltpu.sync_copy(stage_v, deg_sh)

    pltpu.sync_copy(dst2_hbm.at[c, s], idx_v)
    pltpu.sync_copy(ones_hbm, ones_v)
    plsc.subcore_barrier()

    @pl.loop(0, CH2)
    def _(j):
        pltpu.sync_copy(ones_v, deg_sh.at[idx_v.at[j]], add=True)

    plsc.subcore_barrier()

    @pl.when(s == 0)
    def _():
        pltpu.sync_copy(deg_sh, stage_v)
        pltpu.sync_copy(stage_v, deg_out.at[c, 0])


@functools.partial(
    pl.kernel,
    out_type=jax.ShapeDtypeStruct((NC * N, H // 2), jnp.float32),
    mesh=_MESH,
    compiler_params=pltpu.CompilerParams(use_tc_tiling_on_sc=False),
    scratch_types=[
        pltpu.VMEM((CH1, K1), jnp.int32),
        pltpu.VMEM((CH1, K1), jnp.int32),
        pltpu.VMEM((K1, H // 2), jnp.float32),
        pltpu.VMEM((K1, H // 2), jnp.float32),
        pltpu.SemaphoreType.DMA,
        pltpu.SemaphoreType.DMA,
        pltpu.SemaphoreType.DMA,
        pltpu.SemaphoreType.DMA,
        pltpu.VMEM_SHARED((N, H // 2), jnp.float32),
    ],
)
def _layer1_kernel(y_hbm, src1_hbm, dst1_hbm, acc_out, sidx_v, didx_v,
                   r0, r1, g0, g1, s0, s1, acc_sh):
    rows_v = (r0, r1)
    gsem = (g0, g1)
    ssem = (s0, s1)
    c = lax.axis_index("c")
    s = lax.axis_index("s")

    @pl.when(s == 0)
    def _():
        pltpu.sync_copy(y_hbm.at[pl.ds(c * N, N)], acc_sh)

    pltpu.sync_copy(src1_hbm.at[s], sidx_v)
    pltpu.sync_copy(dst1_hbm.at[s], didx_v)
    plsc.subcore_barrier()

    for b in range(NB1):
        pltpu.async_copy(y_hbm.at[pl.ds(c * N, N)].at[sidx_v.at[b]], rows_v[b], gsem[b])

    @pl.loop(0, CH1, step=NB1)
    def _(j):
        for b in range(NB1):
            i = j + b
            pltpu.make_async_copy(y_hbm.at[pl.ds(c * N, N)].at[sidx_v.at[i]], rows_v[b],
                                  gsem[b]).wait()
            pltpu.async_copy(rows_v[b], acc_sh.at[didx_v.at[i]], ssem[b],
                             add=True)

            @pl.when(i + NB1 < CH1)
            def _():
                pltpu.make_async_copy(rows_v[b], acc_sh.at[didx_v.at[i]],
                                      ssem[b]).wait()
                pltpu.async_copy(y_hbm.at[pl.ds(c * N, N)].at[sidx_v.at[i + NB1]], rows_v[b],
                                 gsem[b])

    for b in range(NB1):
        pltpu.make_async_copy(rows_v[b], acc_sh.at[didx_v.at[b]],
                              ssem[b]).wait()

    plsc.subcore_barrier()

    @pl.when(s == 0)
    def _():
        pltpu.sync_copy(acc_sh, acc_out.at[pl.ds(c * N, N)])


@functools.partial(
    pl.kernel,
    out_type=jax.ShapeDtypeStruct((NC * N, C), jnp.float32),
    mesh=_MESH,
    compiler_params=pltpu.CompilerParams(use_tc_tiling_on_sc=False),
    scratch_types=[
        pltpu.VMEM((CH2, K2), jnp.int32),
        pltpu.VMEM((CH2, K2), jnp.int32),
        pltpu.VMEM((K2, C), jnp.float32),
        pltpu.VMEM((K2, C), jnp.float32),
        pltpu.VMEM((K2, C), jnp.float32),
        pltpu.VMEM((K2, C), jnp.float32),
        pltpu.SemaphoreType.DMA,
        pltpu.SemaphoreType.DMA,
        pltpu.SemaphoreType.DMA,
        pltpu.SemaphoreType.DMA,
        pltpu.SemaphoreType.DMA,
        pltpu.SemaphoreType.DMA,
        pltpu.SemaphoreType.DMA,
        pltpu.SemaphoreType.DMA,
        pltpu.VMEM_SHARED((N, C), jnp.float32),
    ],
)
def _layer2_kernel(y2_hbm, src2_hbm, dst2_hbm, acc_out, sidx_v, didx_v,
                   r0, r1, r2, r3, g0, g1, g2, g3, s0, s1, s2, s3, acc_sh):
    rows_v = (r0, r1, r2, r3)
    gsem = (g0, g1, g2, g3)
    ssem = (s0, s1, s2, s3)
    c = lax.axis_index("c")
    s = lax.axis_index("s")

    @pl.when(s == 0)
    def _():
        pltpu.sync_copy(y2_hbm.at[pl.ds(c * N, N)], acc_sh)

    pltpu.sync_copy(src2_hbm.at[c, s], sidx_v)
    pltpu.sync_copy(dst2_hbm.at[c, s], didx_v)
    plsc.subcore_barrier()

    for b in range(NB2):
        pltpu.async_copy(y2_hbm.at[sidx_v.at[b]], rows_v[b], gsem[b])

    @pl.loop(0, CH2, step=NB2)
    def _(j):
        for b in range(NB2):
            i = j + b
            pltpu.make_async_copy(y2_hbm.at[sidx_v.at[i]], rows_v[b],
                                  gsem[b]).wait()
            pltpu.async_copy(rows_v[b], acc_sh.at[didx_v.at[i]], ssem[b],
                             add=True)

            @pl.when(i + NB2 < CH2)
            def _():
                pltpu.make_async_copy(rows_v[b], acc_sh.at[didx_v.at[i]],
                                      ssem[b]).wait()
                pltpu.async_copy(y2_hbm.at[sidx_v.at[i + NB2]], rows_v[b],
                                 gsem[b])

    for b in range(NB2):
        pltpu.make_async_copy(rows_v[b], acc_sh.at[didx_v.at[b]],
                              ssem[b]).wait()

    plsc.subcore_barrier()

    @pl.when(s == 0)
    def _():
        pltpu.sync_copy(acc_sh, acc_out.at[pl.ds(c * N, N)])


def _mm1_body(x_ref, w_ref, degp_ref, y_ref):
    dinv = lax.rsqrt(degp_ref[0] + degp_ref[1] + 1.0)
    xw = jnp.dot(x_ref[...], w_ref[...], preferred_element_type=jnp.float32)
    y_ref[0] = xw[:, : H // 2] * dinv
    y_ref[1] = xw[:, H // 2:] * dinv


def _mm2_body(acc_ref, degp_ref, b1_ref, w2_ref, y2_ref):
    dinv = lax.rsqrt(degp_ref[0] + degp_ref[1] + 1.0)
    h_l = jnp.maximum(acc_ref[0] * dinv + b1_ref[:, : H // 2], 0.0)
    h_r = jnp.maximum(acc_ref[1] * dinv + b1_ref[:, H // 2:], 0.0)
    hw = (jnp.dot(h_l, w2_ref[: H // 2, :], preferred_element_type=jnp.float32)
          + jnp.dot(h_r, w2_ref[H // 2:, :], preferred_element_type=jnp.float32))
    y2 = hw * dinv
    y2_ref[0] = y2
    y2_ref[1] = jnp.zeros_like(y2)


def _final_body(acc_ref, degp_ref, b2_ref, out_ref):
    dinv = lax.rsqrt(degp_ref[0] + degp_ref[1] + 1.0)
    z = (acc_ref[0] + acc_ref[1]) * dinv + b2_ref[:, :]
    m = jnp.max(z, axis=1, keepdims=True)
    zm = z - m
    lse = jnp.log(jnp.sum(jnp.exp(zm), axis=1, keepdims=True))
    out_ref[...] = zm - lse


def kernel(x, edge_index, W1, b1, W2, b2):
    src = edge_index[0].astype(jnp.int32)
    dst = edge_index[1].astype(jnp.int32)

    src1 = src.reshape(NS, CH1, K1)
    dst1 = dst.reshape(NS, CH1, K1)
    src2 = src.reshape(NC, NS, CH2, K2)
    dst2 = dst.reshape(NC, NS, CH2, K2)

    ones_k = jnp.ones((K2,), jnp.float32)
    zeros_n = jnp.zeros((N,), jnp.float32)

    deg_p = _deg_kernel(dst2, ones_k, zeros_n)
    degp_r = deg_p.reshape(NC, N, 1)

    y1 = pl.pallas_call(
        _mm1_body,
        grid=(GR,),
        in_specs=[
            pl.BlockSpec((RB, F), lambda i: (i, 0)),
            pl.BlockSpec((F, H), lambda i: (0, 0)),
            pl.BlockSpec((NC, RB, 1), lambda i: (0, i, 0)),
        ],
        out_specs=pl.BlockSpec((NC, RB, H // 2), lambda i: (0, i, 0)),
        out_shape=jax.ShapeDtypeStruct((NC, N, H // 2), jnp.float32),
    )(x, W1, degp_r)

    acc1 = _layer1_kernel(y1.reshape(NC * N, H // 2), src1, dst1)
    acc1_r = acc1.reshape(NC, N, H // 2)

    y2i = pl.pallas_call(
        _mm2_body,
        grid=(GR,),
        in_specs=[
            pl.BlockSpec((NC, RB, H // 2), lambda i: (0, i, 0)),
            pl.BlockSpec((NC, RB, 1), lambda i: (0, i, 0)),
            pl.BlockSpec((1, H), lambda i: (0, 0)),
            pl.BlockSpec((H, C), lambda i: (0, 0)),
        ],
        out_specs=pl.BlockSpec((NC, RB, C), lambda i: (0, i, 0)),
        out_shape=jax.ShapeDtypeStruct((NC, N, C), jnp.float32),
    )(acc1_r, degp_r, b1.reshape(1, H), W2)

    acc2 = _layer2_kernel(y2i.reshape(NC * N, C), src2, dst2)
    acc2_r = acc2.reshape(NC, N, C)

    out = pl.pallas_call(
        _final_body,
        grid=(GR,),
        in_specs=[
            pl.BlockSpec((NC, RB, C), lambda i: (0, i, 0)),
            pl.BlockSpec((NC, RB, 1), lambda i: (0, i, 0)),
            pl.BlockSpec((1, C), lambda i: (0, 0)),
        ],
        out_specs=pl.BlockSpec((RB, C), lambda i: (i, 0)),
        out_shape=jax.ShapeDtypeStruct((N, C), jnp.float32),
    )(acc2_r, degp_r, b2.reshape(1, C))
    return out


# speedup vs baseline: 23.1875x; 1.0501x over previous
import functools

import jax
import jax.numpy as jnp
from jax import lax
from jax.experimental import pallas as pl
from jax.experimental.pallas import tpu as pltpu
from jax.experimental.pallas import tpu_sc as plsc

N = 10000
E = 160000
F = 256
H = 256
C = 40

NC, NS = 2, 16
RPT = N // NS

K1, CH1 = 100, 100
K2, CH2 = 125, 40

NB1 = 2
NB2 = 4

RB = 1000
GR = N // RB

_MESH = plsc.VectorSubcoreMesh(
    core_axis_name="c", subcore_axis_name="s", num_cores=NC, num_subcores=NS)


@functools.partial(
    pl.kernel,
    out_type=jax.ShapeDtypeStruct((NC, 1, N), jnp.float32),
    mesh=_MESH,
    compiler_params=pltpu.CompilerParams(use_tc_tiling_on_sc=False),
    scratch_types=[
        pltpu.VMEM((CH2, K2), jnp.int32),
        pltpu.VMEM((K2,), jnp.float32),
        pltpu.VMEM((N,), jnp.float32),
        pltpu.VMEM_SHARED((N,), jnp.float32),
    ],
)
def _deg_kernel(dst2_hbm, ones_hbm, zeros_hbm, deg_out, idx_v, ones_v, stage_v,
                deg_sh):
    c = lax.axis_index("c")
    s = lax.axis_index("s")

    @pl.when(s == 0)
    def _():
        pltpu.sync_copy(zeros_hbm, stage_v)
        pltpu.sync_copy(stage_v, deg_sh)

    pltpu.sync_copy(dst2_hbm.at[c, s], idx_v)
    pltpu.sync_copy(ones_hbm, ones_v)
    plsc.subcore_barrier()

    @pl.loop(0, CH2)
    def _(j):
        pltpu.sync_copy(ones_v, deg_sh.at[idx_v.at[j]], add=True)

    plsc.subcore_barrier()

    @pl.when(s == 0)
    def _():
        pltpu.sync_copy(deg_sh, stage_v)
        pltpu.sync_copy(stage_v, deg_out.at[c, 0])


@functools.partial(
    pl.kernel,
    out_type=jax.ShapeDtypeStruct((NC * N, H // 2), jnp.float32),
    mesh=_MESH,
    compiler_params=pltpu.CompilerParams(use_tc_tiling_on_sc=False),
    scratch_types=[
        pltpu.VMEM((CH1, K1), jnp.int32),
        pltpu.VMEM((CH1, K1), jnp.int32),
        pltpu.VMEM((K1, H // 2), jnp.float32),
        pltpu.VMEM((K1, H // 2), jnp.float32),
        pltpu.SemaphoreType.DMA,
        pltpu.SemaphoreType.DMA,
        pltpu.SemaphoreType.DMA,
        pltpu.SemaphoreType.DMA,
        pltpu.VMEM_SHARED((N, H // 2), jnp.float32),
    ],
)
def _layer1_kernel(y_hbm, src1_hbm, dst1_hbm, acc_out, sidx_v, didx_v,
                   r0, r1, g0, g1, s0, s1, acc_sh):
    rows_v = (r0, r1)
    gsem = (g0, g1)
    ssem = (s0, s1)
    c = lax.axis_index("c")
    s = lax.axis_index("s")

    @pl.when(s == 0)
    def _():
        pltpu.sync_copy(y_hbm.at[pl.ds(c * N, N)], acc_sh)

    pltpu.sync_copy(src1_hbm.at[s], sidx_v)
    pltpu.sync_copy(dst1_hbm.at[s], didx_v)
    plsc.subcore_barrier()

    for b in range(NB1):
        pltpu.async_copy(y_hbm.at[pl.ds(c * N, N)].at[sidx_v.at[b]], rows_v[b], gsem[b])

    @pl.loop(0, CH1, step=NB1)
    def _(j):
        for b in range(NB1):
            i = j + b
            pltpu.make_async_copy(y_hbm.at[pl.ds(c * N, N)].at[sidx_v.at[i]], rows_v[b],
                                  gsem[b]).wait()
            pltpu.async_copy(rows_v[b], acc_sh.at[didx_v.at[i]], ssem[b],
                             add=True)

            @pl.when(i + NB1 < CH1)
            def _():
                pltpu.make_async_copy(rows_v[b], acc_sh.at[didx_v.at[i]],
                                      ssem[b]).wait()
                pltpu.async_copy(y_hbm.at[pl.ds(c * N, N)].at[sidx_v.at[i + NB1]], rows_v[b],
                                 gsem[b])

    for b in range(NB1):
        pltpu.make_async_copy(rows_v[b], acc_sh.at[didx_v.at[b]],
                              ssem[b]).wait()

    plsc.subcore_barrier()

    @pl.when(s == 0)
    def _():
        pltpu.sync_copy(acc_sh, acc_out.at[pl.ds(c * N, N)])


@functools.partial(
    pl.kernel,
    out_type=jax.ShapeDtypeStruct((NC * N, C), jnp.float32),
    mesh=_MESH,
    compiler_params=pltpu.CompilerParams(use_tc_tiling_on_sc=False),
    scratch_types=[
        pltpu.VMEM((CH2, K2), jnp.int32),
        pltpu.VMEM((CH2, K2), jnp.int32),
        pltpu.VMEM((K2, C), jnp.float32),
        pltpu.VMEM((K2, C), jnp.float32),
        pltpu.VMEM((K2, C), jnp.float32),
        pltpu.VMEM((K2, C), jnp.float32),
        pltpu.SemaphoreType.DMA,
        pltpu.SemaphoreType.DMA,
        pltpu.SemaphoreType.DMA,
        pltpu.SemaphoreType.DMA,
        pltpu.SemaphoreType.DMA,
        pltpu.SemaphoreType.DMA,
        pltpu.SemaphoreType.DMA,
        pltpu.SemaphoreType.DMA,
        pltpu.VMEM_SHARED((N, C), jnp.float32),
    ],
)
def _layer2_kernel(y2_hbm, src2_hbm, dst2_hbm, acc_out, sidx_v, didx_v,
                   r0, r1, r2, r3, g0, g1, g2, g3, s0, s1, s2, s3, acc_sh):
    rows_v = (r0, r1, r2, r3)
    gsem = (g0, g1, g2, g3)
    ssem = (s0, s1, s2, s3)
    c = lax.axis_index("c")
    s = lax.axis_index("s")

    @pl.when(s == 0)
    def _():
        pltpu.sync_copy(y2_hbm.at[pl.ds(c * N, N)], acc_sh)

    pltpu.sync_copy(src2_hbm.at[c, s], sidx_v)
    pltpu.sync_copy(dst2_hbm.at[c, s], didx_v)
    plsc.subcore_barrier()

    for b in range(NB2):
        pltpu.async_copy(y2_hbm.at[sidx_v.at[b]], rows_v[b], gsem[b])

    @pl.loop(0, CH2, step=NB2)
    def _(j):
        for b in range(NB2):
            i = j + b
            pltpu.make_async_copy(y2_hbm.at[sidx_v.at[i]], rows_v[b],
                                  gsem[b]).wait()
            pltpu.async_copy(rows_v[b], acc_sh.at[didx_v.at[i]], ssem[b],
                             add=True)

            @pl.when(i + NB2 < CH2)
            def _():
                pltpu.make_async_copy(rows_v[b], acc_sh.at[didx_v.at[i]],
                                      ssem[b]).wait()
                pltpu.async_copy(y2_hbm.at[sidx_v.at[i + NB2]], rows_v[b],
                                 gsem[b])

    for b in range(NB2):
        pltpu.make_async_copy(rows_v[b], acc_sh.at[didx_v.at[b]],
                              ssem[b]).wait()

    plsc.subcore_barrier()

    @pl.when(s == 0)
    def _():
        pltpu.sync_copy(acc_sh, acc_out.at[pl.ds(c * N, N)])


def _dinv_bcast(degp_ref):
    deg = degp_ref[0, 0:1, :] + degp_ref[0, 1:2, :] + 1.0
    drow = lax.rsqrt(deg)
    ones = jnp.ones((1, 128), jnp.float32)
    return lax.dot_general(drow, ones, (((0,), (0,)), ((), ())),
                           preferred_element_type=jnp.float32)

def _mm1_body(x_ref, w_ref, degp_ref, y_ref):
    dinv = _dinv_bcast(degp_ref)
    xw = jnp.dot(x_ref[...], w_ref[...], preferred_element_type=jnp.float32)
    y_ref[0] = xw[:, : H // 2] * dinv
    y_ref[1] = xw[:, H // 2:] * dinv


def _mm2_body(acc_ref, degp_ref, b1_ref, w2_ref, y2_ref):
    dinv = _dinv_bcast(degp_ref)
    h_l = jnp.maximum(acc_ref[0] * dinv + b1_ref[:, : H // 2], 0.0)
    h_r = jnp.maximum(acc_ref[1] * dinv + b1_ref[:, H // 2:], 0.0)
    hw = (jnp.dot(h_l, w2_ref[: H // 2, :], preferred_element_type=jnp.float32)
          + jnp.dot(h_r, w2_ref[H // 2:, :], preferred_element_type=jnp.float32))
    y2 = hw * dinv[:, :C]
    y2_ref[0] = y2
    y2_ref[1] = jnp.zeros_like(y2)


def _final_body(acc_ref, degp_ref, b2_ref, out_ref):
    dinv = _dinv_bcast(degp_ref)
    z = (acc_ref[0] + acc_ref[1]) * dinv[:, :C] + b2_ref[:, :]
    m = jnp.max(z, axis=1, keepdims=True)
    zm = z - m
    lse = jnp.log(jnp.sum(jnp.exp(zm), axis=1, keepdims=True))
    out_ref[...] = zm - lse


def kernel(x, edge_index, W1, b1, W2, b2):
    src = edge_index[0].astype(jnp.int32)
    dst = edge_index[1].astype(jnp.int32)

    src1 = src.reshape(NS, CH1, K1)
    dst1 = dst.reshape(NS, CH1, K1)
    src2 = src.reshape(NC, NS, CH2, K2)
    dst2 = dst.reshape(NC, NS, CH2, K2)

    ones_k = jnp.ones((K2,), jnp.float32)
    zeros_n = jnp.zeros((N,), jnp.float32)

    deg_p = _deg_kernel(dst2, ones_k, zeros_n)
    degp_r = deg_p.reshape(NC, GR, RB).transpose(1, 0, 2)

    y1 = pl.pallas_call(
        _mm1_body,
        grid=(GR,),
        in_specs=[
            pl.BlockSpec((RB, F), lambda i: (i, 0)),
            pl.BlockSpec((F, H), lambda i: (0, 0)),
            pl.BlockSpec((1, NC, RB), lambda i: (i, 0, 0)),
        ],
        out_specs=pl.BlockSpec((NC, RB, H // 2), lambda i: (0, i, 0)),
        out_shape=jax.ShapeDtypeStruct((NC, N, H // 2), jnp.float32),
    )(x, W1, degp_r)

    acc1 = _layer1_kernel(y1.reshape(NC * N, H // 2), src1, dst1)
    acc1_r = acc1.reshape(NC, N, H // 2)

    y2i = pl.pallas_call(
        _mm2_body,
        grid=(GR,),
        in_specs=[
            pl.BlockSpec((NC, RB, H // 2), lambda i: (0, i, 0)),
            pl.BlockSpec((1, NC, RB), lambda i: (i, 0, 0)),
            pl.BlockSpec((1, H), lambda i: (0, 0)),
            pl.BlockSpec((H, C), lambda i: (0, 0)),
        ],
        out_specs=pl.BlockSpec((NC, RB, C), lambda i: (0, i, 0)),
        out_shape=jax.ShapeDtypeStruct((NC, N, C), jnp.float32),
    )(acc1_r, degp_r, b1.reshape(1, H), W2)

    acc2 = _layer2_kernel(y2i.reshape(NC * N, C), src2, dst2)
    acc2_r = acc2.reshape(NC, N, C)

    out = pl.pallas_call(
        _final_body,
        grid=(GR,),
        in_specs=[
            pl.BlockSpec((NC, RB, C), lambda i: (0, i, 0)),
            pl.BlockSpec((1, NC, RB), lambda i: (i, 0, 0)),
            pl.BlockSpec((1, C), lambda i: (0, 0)),
        ],
        out_specs=pl.BlockSpec((RB, C), lambda i: (i, 0)),
        out_shape=jax.ShapeDtypeStruct((N, C), jnp.float32),
    )(acc2_r, degp_r, b2.reshape(1, C))
    return out


# speedup vs baseline: 24.9190x; 1.0747x over previous
import functools

import jax
import jax.numpy as jnp
from jax import lax
from jax.experimental import pallas as pl
from jax.experimental.pallas import tpu as pltpu
from jax.experimental.pallas import tpu_sc as plsc

N = 10000
E = 160000
F = 256
H = 256
C = 40

NC, NS = 2, 16
RPT = N // NS

K1, CH1 = 50, 200
K2, CH2 = 125, 40

NB1 = 4
NB2 = 4

RB = 1000
GR = N // RB

_MESH = plsc.VectorSubcoreMesh(
    core_axis_name="c", subcore_axis_name="s", num_cores=NC, num_subcores=NS)


@functools.partial(
    pl.kernel,
    out_type=jax.ShapeDtypeStruct((NC, 1, N), jnp.float32),
    mesh=_MESH,
    compiler_params=pltpu.CompilerParams(use_tc_tiling_on_sc=False),
    scratch_types=[
        pltpu.VMEM((CH2, K2), jnp.int32),
        pltpu.VMEM((K2,), jnp.float32),
        pltpu.VMEM((N,), jnp.float32),
        pltpu.VMEM_SHARED((N,), jnp.float32),
    ],
)
def _deg_kernel(dst2_hbm, ones_hbm, zeros_hbm, deg_out, idx_v, ones_v, stage_v,
                deg_sh):
    c = lax.axis_index("c")
    s = lax.axis_index("s")

    @pl.when(s == 0)
    def _():
        pltpu.sync_copy(zeros_hbm, stage_v)
        pltpu.sync_copy(stage_v, deg_sh)

    pltpu.sync_copy(dst2_hbm.at[c, s], idx_v)
    pltpu.sync_copy(ones_hbm, ones_v)
    plsc.subcore_barrier()

    @pl.loop(0, CH2)
    def _(j):
        pltpu.sync_copy(ones_v, deg_sh.at[idx_v.at[j]], add=True)

    plsc.subcore_barrier()

    @pl.when(s == 0)
    def _():
        pltpu.sync_copy(deg_sh, stage_v)
        pltpu.sync_copy(stage_v, deg_out.at[c, 0])


@functools.partial(
    pl.kernel,
    out_type=jax.ShapeDtypeStruct((NC * N, H // 2), jnp.float32),
    mesh=_MESH,
    compiler_params=pltpu.CompilerParams(use_tc_tiling_on_sc=False),
    scratch_types=[
        pltpu.VMEM((CH1, K1), jnp.int32),
        pltpu.VMEM((CH1, K1), jnp.int32),
        pltpu.VMEM((K1, H // 2), jnp.float32),
        pltpu.VMEM((K1, H // 2), jnp.float32),
        pltpu.VMEM((K1, H // 2), jnp.float32),
        pltpu.VMEM((K1, H // 2), jnp.float32),
        pltpu.SemaphoreType.DMA,
        pltpu.SemaphoreType.DMA,
        pltpu.SemaphoreType.DMA,
        pltpu.SemaphoreType.DMA,
        pltpu.SemaphoreType.DMA,
        pltpu.SemaphoreType.DMA,
        pltpu.SemaphoreType.DMA,
        pltpu.SemaphoreType.DMA,
        pltpu.VMEM_SHARED((N, H // 2), jnp.float32),
    ],
)
def _layer1_kernel(y_hbm, src1_hbm, dst1_hbm, acc_out, sidx_v, didx_v,
                   r0, r1, r2, r3, g0, g1, g2, g3, s0, s1, s2, s3, acc_sh):
    rows_v = (r0, r1, r2, r3)
    gsem = (g0, g1, g2, g3)
    ssem = (s0, s1, s2, s3)
    c = lax.axis_index("c")
    s = lax.axis_index("s")

    @pl.when(s == 0)
    def _():
        pltpu.sync_copy(y_hbm.at[pl.ds(c * N, N)], acc_sh)

    pltpu.sync_copy(src1_hbm.at[s], sidx_v)
    pltpu.sync_copy(dst1_hbm.at[s], didx_v)
    plsc.subcore_barrier()

    for b in range(NB1):
        pltpu.async_copy(y_hbm.at[pl.ds(c * N, N)].at[sidx_v.at[b]], rows_v[b], gsem[b])

    @pl.loop(0, CH1, step=NB1)
    def _(j):
        for b in range(NB1):
            i = j + b
            pltpu.make_async_copy(y_hbm.at[pl.ds(c * N, N)].at[sidx_v.at[i]], rows_v[b],
                                  gsem[b]).wait()
            pltpu.async_copy(rows_v[b], acc_sh.at[didx_v.at[i]], ssem[b],
                             add=True)

            @pl.when(i + NB1 < CH1)
            def _():
                pltpu.make_async_copy(rows_v[b], acc_sh.at[didx_v.at[i]],
                                      ssem[b]).wait()
                pltpu.async_copy(y_hbm.at[pl.ds(c * N, N)].at[sidx_v.at[i + NB1]], rows_v[b],
                                 gsem[b])

    for b in range(NB1):
        pltpu.make_async_copy(rows_v[b], acc_sh.at[didx_v.at[b]],
                              ssem[b]).wait()

    plsc.subcore_barrier()

    @pl.when(s == 0)
    def _():
        pltpu.sync_copy(acc_sh, acc_out.at[pl.ds(c * N, N)])


@functools.partial(
    pl.kernel,
    out_type=jax.ShapeDtypeStruct((NC * N, C), jnp.float32),
    mesh=_MESH,
    compiler_params=pltpu.CompilerParams(use_tc_tiling_on_sc=False),
    scratch_types=[
        pltpu.VMEM((CH2, K2), jnp.int32),
        pltpu.VMEM((CH2, K2), jnp.int32),
        pltpu.VMEM((K2, C), jnp.float32),
        pltpu.VMEM((K2, C), jnp.float32),
        pltpu.VMEM((K2, C), jnp.float32),
        pltpu.VMEM((K2, C), jnp.float32),
        pltpu.SemaphoreType.DMA,
        pltpu.SemaphoreType.DMA,
        pltpu.SemaphoreType.DMA,
        pltpu.SemaphoreType.DMA,
        pltpu.SemaphoreType.DMA,
        pltpu.SemaphoreType.DMA,
        pltpu.SemaphoreType.DMA,
        pltpu.SemaphoreType.DMA,
        pltpu.VMEM_SHARED((N, C), jnp.float32),
    ],
)
def _layer2_kernel(y2_hbm, src2_hbm, dst2_hbm, acc_out, sidx_v, didx_v,
                   r0, r1, r2, r3, g0, g1, g2, g3, s0, s1, s2, s3, acc_sh):
    rows_v = (r0, r1, r2, r3)
    gsem = (g0, g1, g2, g3)
    ssem = (s0, s1, s2, s3)
    c = lax.axis_index("c")
    s = lax.axis_index("s")

    @pl.when(s == 0)
    def _():
        pltpu.sync_copy(y2_hbm.at[pl.ds(c * N, N)], acc_sh)

    pltpu.sync_copy(src2_hbm.at[c, s], sidx_v)
    pltpu.sync_copy(dst2_hbm.at[c, s], didx_v)
    plsc.subcore_barrier()

    for b in range(NB2):
        pltpu.async_copy(y2_hbm.at[sidx_v.at[b]], rows_v[b], gsem[b])

    @pl.loop(0, CH2, step=NB2)
    def _(j):
        for b in range(NB2):
            i = j + b
            pltpu.make_async_copy(y2_hbm.at[sidx_v.at[i]], rows_v[b],
                                  gsem[b]).wait()
            pltpu.async_copy(rows_v[b], acc_sh.at[didx_v.at[i]], ssem[b],
                             add=True)

            @pl.when(i + NB2 < CH2)
            def _():
                pltpu.make_async_copy(rows_v[b], acc_sh.at[didx_v.at[i]],
                                      ssem[b]).wait()
                pltpu.async_copy(y2_hbm.at[sidx_v.at[i + NB2]], rows_v[b],
                                 gsem[b])

    for b in range(NB2):
        pltpu.make_async_copy(rows_v[b], acc_sh.at[didx_v.at[b]],
                              ssem[b]).wait()

    plsc.subcore_barrier()

    @pl.when(s == 0)
    def _():
        pltpu.sync_copy(acc_sh, acc_out.at[pl.ds(c * N, N)])


def _dinv_bcast(degp_ref):
    deg = degp_ref[0, 0:1, :] + degp_ref[0, 1:2, :] + 1.0
    drow = lax.rsqrt(deg)
    ones = jnp.ones((1, 128), jnp.float32)
    return lax.dot_general(drow, ones, (((0,), (0,)), ((), ())),
                           preferred_element_type=jnp.float32)

def _mm1_body(x_ref, w_ref, degp_ref, y_ref):
    dinv = _dinv_bcast(degp_ref)
    xw = jnp.dot(x_ref[...], w_ref[...], preferred_element_type=jnp.float32)
    y_ref[0] = xw[:, : H // 2] * dinv
    y_ref[1] = xw[:, H // 2:] * dinv


def _mm2_body(acc_ref, degp_ref, b1_ref, w2_ref, y2_ref):
    dinv = _dinv_bcast(degp_ref)
    h_l = jnp.maximum(acc_ref[0] * dinv + b1_ref[:, : H // 2], 0.0)
    h_r = jnp.maximum(acc_ref[1] * dinv + b1_ref[:, H // 2:], 0.0)
    hw = (jnp.dot(h_l, w2_ref[: H // 2, :], preferred_element_type=jnp.float32)
          + jnp.dot(h_r, w2_ref[H // 2:, :], preferred_element_type=jnp.float32))
    y2 = hw * dinv[:, :C]
    y2_ref[0] = y2
    y2_ref[1] = jnp.zeros_like(y2)


def _final_body(acc_ref, degp_ref, b2_ref, out_ref):
    dinv = _dinv_bcast(degp_ref)
    z = (acc_ref[0] + acc_ref[1]) * dinv[:, :C] + b2_ref[:, :]
    m = jnp.max(z, axis=1, keepdims=True)
    zm = z - m
    lse = jnp.log(jnp.sum(jnp.exp(zm), axis=1, keepdims=True))
    out_ref[...] = zm - lse


def kernel(x, edge_index, W1, b1, W2, b2):
    src = edge_index[0].astype(jnp.int32)
    dst = edge_index[1].astype(jnp.int32)

    src1 = src.reshape(NS, CH1, K1)
    dst1 = dst.reshape(NS, CH1, K1)
    src2 = src.reshape(NC, NS, CH2, K2)
    dst2 = dst.reshape(NC, NS, CH2, K2)

    ones_k = jnp.ones((K2,), jnp.float32)
    zeros_n = jnp.zeros((N,), jnp.float32)

    deg_p = _deg_kernel(dst2, ones_k, zeros_n)
    degp_r = deg_p.reshape(NC, GR, RB).transpose(1, 0, 2)

    y1 = pl.pallas_call(
        _mm1_body,
        grid=(GR,),
        in_specs=[
            pl.BlockSpec((RB, F), lambda i: (i, 0)),
            pl.BlockSpec((F, H), lambda i: (0, 0)),
            pl.BlockSpec((1, NC, RB), lambda i: (i, 0, 0)),
        ],
        out_specs=pl.BlockSpec((NC, RB, H // 2), lambda i: (0, i, 0)),
        out_shape=jax.ShapeDtypeStruct((NC, N, H // 2), jnp.float32),
    )(x, W1, degp_r)

    acc1 = _layer1_kernel(y1.reshape(NC * N, H // 2), src1, dst1)
    acc1_r = acc1.reshape(NC, N, H // 2)

    y2i = pl.pallas_call(
        _mm2_body,
        grid=(GR,),
        in_specs=[
            pl.BlockSpec((NC, RB, H // 2), lambda i: (0, i, 0)),
            pl.BlockSpec((1, NC, RB), lambda i: (i, 0, 0)),
            pl.BlockSpec((1, H), lambda i: (0, 0)),
            pl.BlockSpec((H, C), lambda i: (0, 0)),
        ],
        out_specs=pl.BlockSpec((NC, RB, C), lambda i: (0, i, 0)),
        out_shape=jax.ShapeDtypeStruct((NC, N, C), jnp.float32),
    )(acc1_r, degp_r, b1.reshape(1, H), W2)

    acc2 = _layer2_kernel(y2i.reshape(NC * N, C), src2, dst2)
    acc2_r = acc2.reshape(NC, N, C)

    out = pl.pallas_call(
        _final_body,
        grid=(GR,),
        in_specs=[
            pl.BlockSpec((NC, RB, C), lambda i: (0, i, 0)),
            pl.BlockSpec((1, NC, RB), lambda i: (i, 0, 0)),
            pl.BlockSpec((1, C), lambda i: (0, 0)),
        ],
        out_specs=pl.BlockSpec((RB, C), lambda i: (i, 0)),
        out_shape=jax.ShapeDtypeStruct((N, C), jnp.float32),
    )(acc2_r, degp_r, b2.reshape(1, C))
    return out
